# baseline (speedup 1.0000x reference)
import functools

import jax
import jax.numpy as jnp
from jax import lax
from jax.experimental import pallas as pl
from jax.experimental.pallas import tpu as pltpu
from jax.experimental.pallas import tpu_sc as plsc

N = 10000
E = 320000
D = 128
DE = 16
ALPHA = 0.5

NC = 2
NS = 16
NW = NC * NS
EPW = E // NW
C = 80
C8 = C // 8
NCHUNK = EPW // C
NINIT = 10
RPS = N // NINIT
E8 = E // 8


def _tables_body(x_ref, ws_ref, wd_ref, ps_ref, pd_ref):
    x = x_ref[...]
    ps_ref[...] = jnp.dot(x, ws_ref[...], preferred_element_type=jnp.float32)
    pd_ref[...] = jnp.dot(x, wd_ref[...], preferred_element_type=jnp.float32)


def _edge_tables(x, ws, wd):
    return pl.pallas_call(
        _tables_body,
        out_shape=(
            jax.ShapeDtypeStruct((N, DE), jnp.float32),
            jax.ShapeDtypeStruct((N, DE), jnp.float32),
        ),
    )(x, ws, wd)


_QBLK = 5000


def _q_body(ea_ref, we_ref, be_ref, q_ref):
    q_ref[...] = (
        jnp.dot(ea_ref[...], we_ref[...], preferred_element_type=jnp.float32)
        + be_ref[...]
    )


def _edge_q(ea_p, we_bd, be_t):
    grid = E8 // _QBLK
    return pl.pallas_call(
        _q_body,
        grid=(grid,),
        in_specs=[
            pl.BlockSpec((_QBLK, D), lambda i: (i, 0)),
            pl.BlockSpec((D, D), lambda i: (0, 0)),
            pl.BlockSpec((1, D), lambda i: (0, 0)),
        ],
        out_specs=pl.BlockSpec((_QBLK, D), lambda i: (i, 0)),
        out_shape=jax.ShapeDtypeStruct((E8, D), jnp.float32),
    )(ea_p, we_bd, be_t.reshape(1, D))


def _node_body(x_ref, part_ref, wx_ref, wa_ref, bn_ref, xo_ref):
    x = x_ref[...]
    agg = part_ref[0] + part_ref[1]
    dx = jnp.dot(x, wx_ref[...], preferred_element_type=jnp.float32)
    dx = dx + jnp.dot(agg, wa_ref[...], preferred_element_type=jnp.float32)
    dx = jnp.maximum(dx + bn_ref[...], 0.0)
    sa = jnp.float32(ALPHA) ** 0.5
    sb = jnp.float32(1.0 - ALPHA) ** 0.5
    xo_ref[...] = sa * dx + sb * x


def _node_update(x, partials, wx, wa, bn):
    return pl.pallas_call(
        _node_body,
        out_shape=jax.ShapeDtypeStruct((N, D), jnp.float32),
    )(x, partials, wx, wa, bn.reshape(1, D))


_BLKE = 6400
_BLK8 = _BLKE // 8


def _qT_body(ea_ref, we_ref, be_ref, q_ref):
    qT = jnp.dot(we_ref[...], ea_ref[...],
                 preferred_element_type=jnp.float32) + be_ref[...]
    q_ref[...] = qT.reshape(DE, _BLK8, 8).transpose(1, 2, 0).reshape(_BLK8, D)


def _edge_qT(eaT, weT, beT):
    return pl.pallas_call(
        _qT_body,
        grid=(E // _BLKE,),
        in_specs=[
            pl.BlockSpec((DE, _BLKE), lambda i: (0, i)),
            pl.BlockSpec((DE, DE), lambda i: (0, 0)),
            pl.BlockSpec((DE, 1), lambda i: (0, 0)),
        ],
        out_specs=pl.BlockSpec((_BLK8, D), lambda i: (i, 0)),
        out_shape=jax.ShapeDtypeStruct((E8, D), jnp.float32),
    )(eaT, weT, beT.reshape(DE, 1))


def _catT_body(a0T_ref, b_ref, c_ref, o_ref, e2_ref):
    def tr(x):
        return x.reshape(_BLK8, 8, DE).transpose(2, 0, 1).reshape(DE, _BLKE)

    o_ref[0:DE, :] = a0T_ref[...]
    o_ref[DE:2 * DE, :] = tr(b_ref[...])
    t = tr(c_ref[...])
    o_ref[2 * DE:3 * DE, :] = t
    e2_ref[...] = t


def _edge_catT(ea0T, b_p, c_p):
    return pl.pallas_call(
        _catT_body,
        grid=(E // _BLKE,),
        in_specs=[
            pl.BlockSpec((DE, _BLKE), lambda i: (0, i)),
            pl.BlockSpec((_BLK8, D), lambda i: (i, 0)),
            pl.BlockSpec((_BLK8, D), lambda i: (i, 0)),
        ],
        out_specs=(pl.BlockSpec((3 * DE, _BLKE), lambda i: (0, i)),
                   pl.BlockSpec((DE, _BLKE), lambda i: (0, i))),
        out_shape=(jax.ShapeDtypeStruct((3 * DE, E), jnp.float32),
                   jax.ShapeDtypeStruct((DE, E), jnp.float32)),
    )(ea0T, b_p, c_p)


def _sc_edge_body(ps_hbm, pd_hbm, q_hbm, src_hbm, dst_hbm, zeros_hbm,
                  outs, idx_s, idx_d, q_v, rows_s, rows_d, out_v, out_p,
                  agg_sp, semf, semg, sems):
    eap_hbm, part_hbm = outs
    cid = lax.axis_index("c")
    sid = lax.axis_index("s")
    wid = sid * NC + cid

    @pl.when(sid < NINIT)
    def _init():
        pltpu.sync_copy(zeros_hbm.at[pl.ds(sid * RPS, RPS)],
                        agg_sp.at[pl.ds(sid * RPS, RPS)])

    plsc.subcore_barrier()
    base0 = wid * EPW

    def front(c, fb):
        base = base0 + c * C
        pltpu.async_copy(src_hbm.at[pl.ds(base, C)], idx_s.at[fb], semf[fb])
        pltpu.async_copy(dst_hbm.at[pl.ds(base, C)], idx_d.at[fb], semf[fb])
        pltpu.async_copy(q_hbm.at[pl.ds(base // 8, C8)], q_v.at[fb], semf[fb])

    def wait_front(fb):
        pltpu.make_async_copy(src_hbm.at[pl.ds(0, C)], idx_s.at[fb],
                              semf[fb]).wait()
        pltpu.make_async_copy(dst_hbm.at[pl.ds(0, C)], idx_d.at[fb],
                              semf[fb]).wait()
        pltpu.make_async_copy(q_hbm.at[pl.ds(0, C8)], q_v.at[fb],
                              semf[fb]).wait()

    def gathers(fb, rb):
        pltpu.async_copy(ps_hbm.at[idx_s.at[fb]], rows_s.at[rb], semg[rb])
        pltpu.async_copy(pd_hbm.at[idx_d.at[fb]], rows_d.at[rb], semg[rb])

    def wait_gathers(fb, rb):
        pltpu.make_async_copy(ps_hbm.at[idx_s.at[fb]], rows_s.at[rb],
                              semg[rb]).wait()
        pltpu.make_async_copy(pd_hbm.at[idx_d.at[fb]], rows_d.at[rb],
                              semg[rb]).wait()

    def compute(fb, rb):
        def block(i, carry):
            for j in range(8):
                e = i * 8 + j
                v = rows_s[rb, e] + rows_d[rb, e] + q_v[fb, i, pl.ds(j * DE, DE)]
                v = jnp.maximum(v, 0.0)
                out_v[rb, e] = v
                out_p[rb, i, pl.ds(j * DE, DE)] = v
            return carry

        lax.fori_loop(0, C8, block, 0)

    def stores(c, fb, rb):
        base = base0 + c * C
        pltpu.async_copy(out_p.at[rb], eap_hbm.at[pl.ds(base // 8, C8)],
                         sems[rb])
        pltpu.sync_copy(out_v.at[rb], agg_sp.at[idx_d.at[fb]], add=True)

    def wait_stores(fb, rb):
        pltpu.make_async_copy(out_p.at[rb], eap_hbm.at[pl.ds(0, C8)],
                              sems[rb]).wait()

    def step(c, fb, rb, do_wait_stores, do_next_gather, do_front2):
        if do_next_gather:
            wait_front((fb + 1) % 4)
            gathers((fb + 1) % 4, rb ^ 1)
        if do_wait_stores:
            wait_stores(fb, rb)
        wait_gathers(fb, rb)
        compute(fb, rb)
        stores(c, fb, rb)
        if do_front2:
            front(c + 2, (fb + 2) % 4)

    front(0, 0)
    front(1, 1)
    wait_front(0)
    gathers(0, 0)
    step(0, 0, 0, False, True, True)
    step(1, 1, 1, False, True, True)

    def group(g, carry):
        c0 = 2 + g * 4
        for p in range(4):
            step(c0 + p, (2 + p) % 4, p % 2, True, True, True)
        return carry

    lax.fori_loop(0, (NCHUNK - 5) // 4, group, 0)

    c = NCHUNK - 3
    step(c, c % 4, c % 2, True, True, True)
    step(c + 1, (c + 1) % 4, (c + 1) % 2, True, True, False)
    step(c + 2, (c + 2) % 4, (c + 2) % 2, True, False, False)
    wait_stores((c + 1) % 4, (c + 1) % 2)
    wait_stores((c + 2) % 4, (c + 2) % 2)

    plsc.subcore_barrier()

    @pl.when(sid < NINIT)
    def _drain():
        pltpu.sync_copy(agg_sp.at[pl.ds(sid * RPS, RPS)],
                        part_hbm.at[cid].at[pl.ds(sid * RPS, RPS)])


def _sc_wrap():
    def body(ps, pd, q, src, dst, zeros, *rest):
        outs = rest[:2]
        (idx_s, idx_d, q_v, rows_s, rows_d, out_v, out_p, agg_sp) = rest[2:10]
        sems = rest[10:]
        _sc_edge_body(ps, pd, q, src, dst, zeros, outs,
                      idx_s, idx_d, q_v, rows_s, rows_d, out_v, out_p,
                      agg_sp, sems[0:4], sems[4:6], sems[6:8])
    return body


_SC_PARAMS = pltpu.CompilerParams(use_tc_tiling_on_sc=False)
_SC_SCRATCH = [
    pltpu.VMEM((4, C), jnp.int32),
    pltpu.VMEM((4, C), jnp.int32),
    pltpu.VMEM((4, C8, D), jnp.float32),
    pltpu.VMEM((2, C, DE), jnp.float32),
    pltpu.VMEM((2, C, DE), jnp.float32),
    pltpu.VMEM((2, C, DE), jnp.float32),
    pltpu.VMEM((2, C8, D), jnp.float32),
    pltpu.VMEM_SHARED((N, DE), jnp.float32),
] + [pltpu.SemaphoreType.DMA] * 8


@functools.lru_cache(maxsize=None)
def _sc_l1_call():
    mesh = plsc.VectorSubcoreMesh(core_axis_name="c", subcore_axis_name="s")
    return pl.kernel(
        _sc_wrap(),
        out_type=(
            jax.ShapeDtypeStruct((E8, D), jnp.float32),
            jax.ShapeDtypeStruct((NC, N, DE), jnp.float32),
        ),
        mesh=mesh,
        compiler_params=_SC_PARAMS,
        scratch_types=list(_SC_SCRATCH),
    )


def kernel(x, edge_index, edge_attr, We1, be1, Wn1, bn1, We2, be2, Wn2, bn2):
    src = edge_index[0]
    dst = edge_index[1]
    zeros = jnp.zeros((N, DE), jnp.float32)
    eye8 = jnp.eye(8, dtype=jnp.float32)
    sc_call = _sc_l1_call()

    ea0T = edge_attr.T

    ps, pd = _edge_tables(x, We1[:D], We1[D:2 * D])
    q_p = _edge_qT(ea0T, We1[2 * D:].T, be1)
    ea1_p, partials = sc_call(ps, pd, q_p, src, dst, zeros)
    x = _node_update(x, partials, Wn1[:D], Wn1[D:], bn1)

    ps, pd = _edge_tables(x, We2[:D], We2[D:2 * D])
    q_p = _edge_q(ea1_p, jnp.kron(eye8, We2[2 * D:]), jnp.tile(be2, 8))
    ea2_p, partials = sc_call(ps, pd, q_p, src, dst, zeros)
    x = _node_update(x, partials, Wn2[:D], Wn2[D:], bn2)

    catT, ea2T = _edge_catT(ea0T, ea1_p, ea2_p)
    return x, ea2T.T, catT.T


# speedup vs baseline: 1.2116x; 1.2116x over previous
import functools

import jax
import jax.numpy as jnp
from jax import lax
from jax.experimental import pallas as pl
from jax.experimental.pallas import tpu as pltpu
from jax.experimental.pallas import tpu_sc as plsc

N = 10000
E = 320000
D = 128
DE = 16
ALPHA = 0.5

NC = 2
NS = 16
NW = NC * NS
EPW = E // NW
C = 80
C8 = C // 8
NCHUNK = EPW // C
NINIT = 10
RPS = N // NINIT
E8 = E // 8


def _tables_body(x_ref, ws_ref, wd_ref, ps_ref, pd_ref):
    x = x_ref[...]
    ps_ref[...] = jnp.dot(x, ws_ref[...], preferred_element_type=jnp.float32)
    pd_ref[...] = jnp.dot(x, wd_ref[...], preferred_element_type=jnp.float32)


def _edge_tables(x, ws, wd):
    return pl.pallas_call(
        _tables_body,
        out_shape=(
            jax.ShapeDtypeStruct((N, DE), jnp.float32),
            jax.ShapeDtypeStruct((N, DE), jnp.float32),
        ),
    )(x, ws, wd)


_QBLK = 5000


def _q_body(ea_ref, we_ref, be_ref, q_ref):
    q_ref[...] = (
        jnp.dot(ea_ref[...], we_ref[...], preferred_element_type=jnp.float32)
        + be_ref[...]
    )


def _edge_q(ea_p, we_bd, be_t):
    grid = E8 // _QBLK
    return pl.pallas_call(
        _q_body,
        grid=(grid,),
        in_specs=[
            pl.BlockSpec((_QBLK, D), lambda i: (i, 0)),
            pl.BlockSpec((D, D), lambda i: (0, 0)),
            pl.BlockSpec((1, D), lambda i: (0, 0)),
        ],
        out_specs=pl.BlockSpec((_QBLK, D), lambda i: (i, 0)),
        out_shape=jax.ShapeDtypeStruct((E8, D), jnp.float32),
    )(ea_p, we_bd, be_t.reshape(1, D))


def _node_body(x_ref, part_ref, wx_ref, wa_ref, bn_ref, xo_ref):
    x = x_ref[...]
    agg = part_ref[0] + part_ref[1]
    dx = jnp.dot(x, wx_ref[...], preferred_element_type=jnp.float32)
    dx = dx + jnp.dot(agg, wa_ref[...], preferred_element_type=jnp.float32)
    dx = jnp.maximum(dx + bn_ref[...], 0.0)
    sa = jnp.float32(ALPHA) ** 0.5
    sb = jnp.float32(1.0 - ALPHA) ** 0.5
    xo_ref[...] = sa * dx + sb * x


def _node_update(x, partials, wx, wa, bn):
    return pl.pallas_call(
        _node_body,
        out_shape=jax.ShapeDtypeStruct((N, D), jnp.float32),
    )(x, partials, wx, wa, bn.reshape(1, D))


_CATBLK = 4000


def _cat_body(a_ref, b_ref, c_ref, o_ref):
    for j in range(8):
        o_ref[:, 48 * j:48 * j + 16] = a_ref[:, 16 * j:16 * j + 16]
        o_ref[:, 48 * j + 16:48 * j + 32] = b_ref[:, 16 * j:16 * j + 16]
        o_ref[:, 48 * j + 32:48 * j + 48] = c_ref[:, 16 * j:16 * j + 16]


def _edge_cat(a_p, b_p, c_p):
    grid = E8 // _CATBLK
    return pl.pallas_call(
        _cat_body,
        grid=(grid,),
        in_specs=[
            pl.BlockSpec((_CATBLK, D), lambda i: (i, 0)),
            pl.BlockSpec((_CATBLK, D), lambda i: (i, 0)),
            pl.BlockSpec((_CATBLK, D), lambda i: (i, 0)),
        ],
        out_specs=pl.BlockSpec((_CATBLK, 3 * D), lambda i: (i, 0)),
        out_shape=jax.ShapeDtypeStruct((E8, 3 * D), jnp.float32),
    )(a_p, b_p, c_p)


def _sc_edge_body(layer2, ps_hbm, pd_hbm, q_hbm, src_hbm, dst_hbm, zeros_hbm,
                  outs, idx_s, idx_d, q_v, rows_s, rows_d, out_v, out_p,
                  agg_sp, semf, semg, sems):
    if layer2:
        ea_hbm, eap_hbm, part_hbm = outs
    else:
        eap_hbm, part_hbm = outs
    cid = lax.axis_index("c")
    sid = lax.axis_index("s")
    wid = sid * NC + cid

    @pl.when(sid < NINIT)
    def _init():
        pltpu.sync_copy(zeros_hbm.at[pl.ds(sid * RPS, RPS)],
                        agg_sp.at[pl.ds(sid * RPS, RPS)])

    plsc.subcore_barrier()
    base0 = wid * EPW

    def front(c, fb):
        base = base0 + c * C
        pltpu.async_copy(src_hbm.at[pl.ds(base, C)], idx_s.at[fb], semf[fb])
        pltpu.async_copy(dst_hbm.at[pl.ds(base, C)], idx_d.at[fb], semf[fb])
        pltpu.async_copy(q_hbm.at[pl.ds(base // 8, C8)], q_v.at[fb], semf[fb])

    def wait_front(fb):
        pltpu.make_async_copy(src_hbm.at[pl.ds(0, C)], idx_s.at[fb],
                              semf[fb]).wait()
        pltpu.make_async_copy(dst_hbm.at[pl.ds(0, C)], idx_d.at[fb],
                              semf[fb]).wait()
        pltpu.make_async_copy(q_hbm.at[pl.ds(0, C8)], q_v.at[fb],
                              semf[fb]).wait()

    def gathers(fb, rb):
        pltpu.async_copy(ps_hbm.at[idx_s.at[fb]], rows_s.at[rb], semg[rb])
        pltpu.async_copy(pd_hbm.at[idx_d.at[fb]], rows_d.at[rb], semg[rb])

    def wait_gathers(fb, rb):
        pltpu.make_async_copy(ps_hbm.at[idx_s.at[fb]], rows_s.at[rb],
                              semg[rb]).wait()
        pltpu.make_async_copy(pd_hbm.at[idx_d.at[fb]], rows_d.at[rb],
                              semg[rb]).wait()

    def compute(fb, rb):
        def block(i, carry):
            for j in range(8):
                e = i * 8 + j
                v = rows_s[rb, e] + rows_d[rb, e] + q_v[fb, i, pl.ds(j * DE, DE)]
                v = jnp.maximum(v, 0.0)
                out_v[rb, e] = v
                out_p[rb, i, pl.ds(j * DE, DE)] = v
            return carry

        lax.fori_loop(0, C8, block, 0)

    def stores(c, fb, rb):
        base = base0 + c * C
        pltpu.async_copy(out_p.at[rb], eap_hbm.at[pl.ds(base // 8, C8)],
                         sems[rb])
        if layer2:
            pltpu.async_copy(out_v.at[rb], ea_hbm.at[pl.ds(base, C)], sems[rb])
        pltpu.sync_copy(out_v.at[rb], agg_sp.at[idx_d.at[fb]], add=True)

    def wait_stores(fb, rb):
        pltpu.make_async_copy(out_p.at[rb], eap_hbm.at[pl.ds(0, C8)],
                              sems[rb]).wait()
        if layer2:
            pltpu.make_async_copy(out_v.at[rb], ea_hbm.at[pl.ds(0, C)],
                                  sems[rb]).wait()

    def step(c, fb, rb, do_wait_stores, do_next_gather, do_front2):
        if do_next_gather:
            wait_front((fb + 1) % 4)
            gathers((fb + 1) % 4, rb ^ 1)
        if do_wait_stores:
            wait_stores(fb, rb)
        wait_gathers(fb, rb)
        compute(fb, rb)
        stores(c, fb, rb)
        if do_front2:
            front(c + 2, (fb + 2) % 4)

    front(0, 0)
    front(1, 1)
    wait_front(0)
    gathers(0, 0)
    step(0, 0, 0, False, True, True)
    step(1, 1, 1, False, True, True)

    def group(g, carry):
        c0 = 2 + g * 4
        for p in range(4):
            step(c0 + p, (2 + p) % 4, p % 2, True, True, True)
        return carry

    lax.fori_loop(0, (NCHUNK - 5) // 4, group, 0)

    c = NCHUNK - 3
    step(c, c % 4, c % 2, True, True, True)
    step(c + 1, (c + 1) % 4, (c + 1) % 2, True, True, False)
    step(c + 2, (c + 2) % 4, (c + 2) % 2, True, False, False)
    wait_stores((c + 1) % 4, (c + 1) % 2)
    wait_stores((c + 2) % 4, (c + 2) % 2)

    plsc.subcore_barrier()

    @pl.when(sid < NINIT)
    def _drain():
        pltpu.sync_copy(agg_sp.at[pl.ds(sid * RPS, RPS)],
                        part_hbm.at[cid].at[pl.ds(sid * RPS, RPS)])


def _sc_wrap(layer2):
    def body(ps, pd, q, src, dst, zeros, *rest):
        n_out = 3 if layer2 else 2
        outs = rest[:n_out]
        (idx_s, idx_d, q_v, rows_s, rows_d, out_v, out_p, agg_sp) = \
            rest[n_out:n_out + 8]
        sems = rest[n_out + 8:]
        _sc_edge_body(layer2, ps, pd, q, src, dst, zeros, outs,
                      idx_s, idx_d, q_v, rows_s, rows_d, out_v, out_p,
                      agg_sp, sems[0:4], sems[4:6], sems[6:8])
    return body


_SC_PARAMS = pltpu.CompilerParams(use_tc_tiling_on_sc=False)
_SC_SCRATCH = [
    pltpu.VMEM((4, C), jnp.int32),
    pltpu.VMEM((4, C), jnp.int32),
    pltpu.VMEM((4, C8, D), jnp.float32),
    pltpu.VMEM((2, C, DE), jnp.float32),
    pltpu.VMEM((2, C, DE), jnp.float32),
    pltpu.VMEM((2, C, DE), jnp.float32),
    pltpu.VMEM((2, C8, D), jnp.float32),
    pltpu.VMEM_SHARED((N, DE), jnp.float32),
] + [pltpu.SemaphoreType.DMA] * 8


@functools.lru_cache(maxsize=None)
def _sc_l1_call():
    mesh = plsc.VectorSubcoreMesh(core_axis_name="c", subcore_axis_name="s")
    return pl.kernel(
        _sc_wrap(False),
        out_type=(
            jax.ShapeDtypeStruct((E8, D), jnp.float32),
            jax.ShapeDtypeStruct((NC, N, DE), jnp.float32),
        ),
        mesh=mesh,
        compiler_params=_SC_PARAMS,
        scratch_types=list(_SC_SCRATCH),
    )


@functools.lru_cache(maxsize=None)
def _sc_l2_call():
    mesh = plsc.VectorSubcoreMesh(core_axis_name="c", subcore_axis_name="s")
    return pl.kernel(
        _sc_wrap(True),
        out_type=(
            jax.ShapeDtypeStruct((E, DE), jnp.float32),
            jax.ShapeDtypeStruct((E8, D), jnp.float32),
            jax.ShapeDtypeStruct((NC, N, DE), jnp.float32),
        ),
        mesh=mesh,
        compiler_params=_SC_PARAMS,
        scratch_types=list(_SC_SCRATCH),
    )


def kernel(x, edge_index, edge_attr, We1, be1, Wn1, bn1, We2, be2, Wn2, bn2):
    src = edge_index[0]
    dst = edge_index[1]
    zeros = jnp.zeros((N, DE), jnp.float32)
    eye8 = jnp.eye(8, dtype=jnp.float32)

    ea0_p = edge_attr.reshape(E8, D)

    ps, pd = _edge_tables(x, We1[:D], We1[D:2 * D])
    q_p = _edge_q(ea0_p, jnp.kron(eye8, We1[2 * D:]), jnp.tile(be1, 8))
    ea1_p, partials = _sc_l1_call()(ps, pd, q_p, src, dst, zeros)
    x = _node_update(x, partials, Wn1[:D], Wn1[D:], bn1)

    ps, pd = _edge_tables(x, We2[:D], We2[D:2 * D])
    q_p = _edge_q(ea1_p, jnp.kron(eye8, We2[2 * D:]), jnp.tile(be2, 8))
    ea2, ea2_p, partials = _sc_l2_call()(ps, pd, q_p, src, dst, zeros)
    x = _node_update(x, partials, Wn2[:D], Wn2[D:], bn2)

    cat = _edge_cat(ea0_p, ea1_p, ea2_p)
    return x, ea2, cat.reshape(E, 3 * DE)


# speedup vs baseline: 1.2384x; 1.0221x over previous
import functools

import jax
import jax.numpy as jnp
from jax import lax
from jax.experimental import pallas as pl
from jax.experimental.pallas import tpu as pltpu
from jax.experimental.pallas import tpu_sc as plsc

N = 10000
E = 320000
D = 128
DE = 16
ALPHA = 0.5

NC = 2
NS = 16
NW = NC * NS
EPW = E // NW
C = 80
C8 = C // 8
NCHUNK = EPW // C
NINIT = 10
RPS = N // NINIT
E8 = E // 8


def _tables_body(x_ref, ws_ref, wd_ref, ps_ref, pd_ref):
    x = x_ref[...]
    ps_ref[...] = jnp.dot(x, ws_ref[...], preferred_element_type=jnp.float32)
    pd_ref[...] = jnp.dot(x, wd_ref[...], preferred_element_type=jnp.float32)


def _edge_tables(x, ws, wd):
    return pl.pallas_call(
        _tables_body,
        out_shape=(
            jax.ShapeDtypeStruct((N, DE), jnp.float32),
            jax.ShapeDtypeStruct((N, DE), jnp.float32),
        ),
    )(x, ws, wd)


_QBLK = 5000


def _q_body(ea_ref, we_ref, be_ref, q_ref):
    q_ref[...] = (
        jnp.dot(ea_ref[...], we_ref[...], preferred_element_type=jnp.float32)
        + be_ref[...]
    )


def _edge_q(ea_p, we_bd, be_t):
    grid = E8 // _QBLK
    return pl.pallas_call(
        _q_body,
        grid=(grid,),
        in_specs=[
            pl.BlockSpec((_QBLK, D), lambda i: (i, 0)),
            pl.BlockSpec((D, D), lambda i: (0, 0)),
            pl.BlockSpec((1, D), lambda i: (0, 0)),
        ],
        out_specs=pl.BlockSpec((_QBLK, D), lambda i: (i, 0)),
        out_shape=jax.ShapeDtypeStruct((E8, D), jnp.float32),
    )(ea_p, we_bd, be_t.reshape(1, D))


def _node_body(x_ref, part_ref, wx_ref, wa_ref, bn_ref, xo_ref):
    x = x_ref[...]
    agg = part_ref[0] + part_ref[1]
    dx = jnp.dot(x, wx_ref[...], preferred_element_type=jnp.float32)
    dx = dx + jnp.dot(agg, wa_ref[...], preferred_element_type=jnp.float32)
    dx = jnp.maximum(dx + bn_ref[...], 0.0)
    sa = jnp.float32(ALPHA) ** 0.5
    sb = jnp.float32(1.0 - ALPHA) ** 0.5
    xo_ref[...] = sa * dx + sb * x


def _node_update(x, partials, wx, wa, bn):
    return pl.pallas_call(
        _node_body,
        out_shape=jax.ShapeDtypeStruct((N, D), jnp.float32),
    )(x, partials, wx, wa, bn.reshape(1, D))


_CATBLK = 4000


def _cat_body(a_ref, b_ref, c_ref, o_ref):
    for j in range(8):
        o_ref[:, 48 * j:48 * j + 16] = a_ref[:, 16 * j:16 * j + 16]
        o_ref[:, 48 * j + 16:48 * j + 32] = b_ref[:, 16 * j:16 * j + 16]
        o_ref[:, 48 * j + 32:48 * j + 48] = c_ref[:, 16 * j:16 * j + 16]


def _edge_cat(a_p, b_p, c_p):
    grid = E8 // _CATBLK
    return pl.pallas_call(
        _cat_body,
        grid=(grid,),
        in_specs=[
            pl.BlockSpec((_CATBLK, D), lambda i: (i, 0)),
            pl.BlockSpec((_CATBLK, D), lambda i: (i, 0)),
            pl.BlockSpec((_CATBLK, D), lambda i: (i, 0)),
        ],
        out_specs=pl.BlockSpec((_CATBLK, 3 * D), lambda i: (i, 0)),
        out_shape=jax.ShapeDtypeStruct((E8, 3 * D), jnp.float32),
    )(a_p, b_p, c_p)


def _sc_edge_body(layer2, ps_hbm, pd_hbm, q_hbm, src_hbm, dst_hbm, zeros_hbm,
                  outs, idx_s, idx_d, q_v, rows_s, rows_d, out_v, out_p,
                  agg_sp, semf, semg, sems, semsc):
    if layer2:
        ea_hbm, eap_hbm, part_hbm = outs
    else:
        eap_hbm, part_hbm = outs
    cid = lax.axis_index("c")
    sid = lax.axis_index("s")
    wid = sid * NC + cid

    @pl.when(sid < NINIT)
    def _init():
        pltpu.sync_copy(zeros_hbm.at[pl.ds(sid * RPS, RPS)],
                        agg_sp.at[pl.ds(sid * RPS, RPS)])

    plsc.subcore_barrier()
    base0 = wid * EPW

    def front(c, fb):
        base = base0 + c * C
        pltpu.async_copy(src_hbm.at[pl.ds(base, C)], idx_s.at[fb], semf[fb])
        pltpu.async_copy(dst_hbm.at[pl.ds(base, C)], idx_d.at[fb], semf[fb])
        pltpu.async_copy(q_hbm.at[pl.ds(base // 8, C8)], q_v.at[fb], semf[fb])

    def wait_front(fb):
        pltpu.make_async_copy(src_hbm.at[pl.ds(0, C)], idx_s.at[fb],
                              semf[fb]).wait()
        pltpu.make_async_copy(dst_hbm.at[pl.ds(0, C)], idx_d.at[fb],
                              semf[fb]).wait()
        pltpu.make_async_copy(q_hbm.at[pl.ds(0, C8)], q_v.at[fb],
                              semf[fb]).wait()

    def gathers(fb, rb):
        pltpu.async_copy(ps_hbm.at[idx_s.at[fb]], rows_s.at[rb], semg[rb])
        pltpu.async_copy(pd_hbm.at[idx_d.at[fb]], rows_d.at[rb], semg[rb])

    def wait_gathers(fb, rb):
        pltpu.make_async_copy(ps_hbm.at[idx_s.at[fb]], rows_s.at[rb],
                              semg[rb]).wait()
        pltpu.make_async_copy(pd_hbm.at[idx_d.at[fb]], rows_d.at[rb],
                              semg[rb]).wait()

    def compute(fb, rb):
        def block(i, carry):
            for j in range(8):
                e = i * 8 + j
                v = rows_s[rb, e] + rows_d[rb, e] + q_v[fb, i, pl.ds(j * DE, DE)]
                v = jnp.maximum(v, 0.0)
                out_v[rb, e] = v
                out_p[rb, i, pl.ds(j * DE, DE)] = v
            return carry

        lax.fori_loop(0, C8, block, 0)

    def stores(c, fb, rb):
        base = base0 + c * C
        pltpu.async_copy(out_p.at[rb], eap_hbm.at[pl.ds(base // 8, C8)],
                         sems[rb])
        if layer2:
            pltpu.async_copy(out_v.at[rb], ea_hbm.at[pl.ds(base, C)], sems[rb])
        pltpu.async_copy(out_v.at[rb], agg_sp.at[idx_d.at[fb]], semsc[rb],
                         add=True)

    def wait_stores(fb, rb):
        pltpu.make_async_copy(out_p.at[rb], eap_hbm.at[pl.ds(0, C8)],
                              sems[rb]).wait()
        if layer2:
            pltpu.make_async_copy(out_v.at[rb], ea_hbm.at[pl.ds(0, C)],
                                  sems[rb]).wait()
        pltpu.make_async_copy(out_v.at[rb], agg_sp.at[idx_d.at[fb]],
                              semsc[rb]).wait()

    def step(c, fb, rb, do_wait_stores, do_next_gather, do_front2):
        if do_next_gather:
            wait_front((fb + 1) % 4)
            gathers((fb + 1) % 4, rb ^ 1)
        if do_wait_stores:
            wait_stores(fb, rb)
        wait_gathers(fb, rb)
        compute(fb, rb)
        stores(c, fb, rb)
        if do_front2:
            front(c + 2, (fb + 2) % 4)

    front(0, 0)
    front(1, 1)
    wait_front(0)
    gathers(0, 0)
    step(0, 0, 0, False, True, True)
    step(1, 1, 1, False, True, True)

    def group(g, carry):
        c0 = 2 + g * 4
        for p in range(4):
            step(c0 + p, (2 + p) % 4, p % 2, True, True, True)
        return carry

    lax.fori_loop(0, (NCHUNK - 5) // 4, group, 0)

    c = NCHUNK - 3
    step(c, c % 4, c % 2, True, True, True)
    step(c + 1, (c + 1) % 4, (c + 1) % 2, True, True, False)
    step(c + 2, (c + 2) % 4, (c + 2) % 2, True, False, False)
    wait_stores((c + 1) % 4, (c + 1) % 2)
    wait_stores((c + 2) % 4, (c + 2) % 2)

    plsc.subcore_barrier()

    @pl.when(sid < NINIT)
    def _drain():
        pltpu.sync_copy(agg_sp.at[pl.ds(sid * RPS, RPS)],
                        part_hbm.at[cid].at[pl.ds(sid * RPS, RPS)])


def _sc_wrap(layer2):
    def body(ps, pd, q, src, dst, zeros, *rest):
        n_out = 3 if layer2 else 2
        outs = rest[:n_out]
        (idx_s, idx_d, q_v, rows_s, rows_d, out_v, out_p, agg_sp) = \
            rest[n_out:n_out + 8]
        sems = rest[n_out + 8:]
        _sc_edge_body(layer2, ps, pd, q, src, dst, zeros, outs,
                      idx_s, idx_d, q_v, rows_s, rows_d, out_v, out_p,
                      agg_sp, sems[0:4], sems[4:6], sems[6:8], sems[8:10])
    return body


_SC_PARAMS = pltpu.CompilerParams(use_tc_tiling_on_sc=False)
_SC_SCRATCH = [
    pltpu.VMEM((4, C), jnp.int32),
    pltpu.VMEM((4, C), jnp.int32),
    pltpu.VMEM((4, C8, D), jnp.float32),
    pltpu.VMEM((2, C, DE), jnp.float32),
    pltpu.VMEM((2, C, DE), jnp.float32),
    pltpu.VMEM((2, C, DE), jnp.float32),
    pltpu.VMEM((2, C8, D), jnp.float32),
    pltpu.VMEM_SHARED((N, DE), jnp.float32),
] + [pltpu.SemaphoreType.DMA] * 10


@functools.lru_cache(maxsize=None)
def _sc_l1_call():
    mesh = plsc.VectorSubcoreMesh(core_axis_name="c", subcore_axis_name="s")
    return pl.kernel(
        _sc_wrap(False),
        out_type=(
            jax.ShapeDtypeStruct((E8, D), jnp.float32),
            jax.ShapeDtypeStruct((NC, N, DE), jnp.float32),
        ),
        mesh=mesh,
        compiler_params=_SC_PARAMS,
        scratch_types=list(_SC_SCRATCH),
    )


@functools.lru_cache(maxsize=None)
def _sc_l2_call():
    mesh = plsc.VectorSubcoreMesh(core_axis_name="c", subcore_axis_name="s")
    return pl.kernel(
        _sc_wrap(True),
        out_type=(
            jax.ShapeDtypeStruct((E, DE), jnp.float32),
            jax.ShapeDtypeStruct((E8, D), jnp.float32),
            jax.ShapeDtypeStruct((NC, N, DE), jnp.float32),
        ),
        mesh=mesh,
        compiler_params=_SC_PARAMS,
        scratch_types=list(_SC_SCRATCH),
    )


def kernel(x, edge_index, edge_attr, We1, be1, Wn1, bn1, We2, be2, Wn2, bn2):
    src = edge_index[0]
    dst = edge_index[1]
    zeros = jnp.zeros((N, DE), jnp.float32)
    eye8 = jnp.eye(8, dtype=jnp.float32)

    ea0_p = edge_attr.reshape(E8, D)

    ps, pd = _edge_tables(x, We1[:D], We1[D:2 * D])
    q_p = _edge_q(ea0_p, jnp.kron(eye8, We1[2 * D:]), jnp.tile(be1, 8))
    ea1_p, partials = _sc_l1_call()(ps, pd, q_p, src, dst, zeros)
    x = _node_update(x, partials, Wn1[:D], Wn1[D:], bn1)

    ps, pd = _edge_tables(x, We2[:D], We2[D:2 * D])
    q_p = _edge_q(ea1_p, jnp.kron(eye8, We2[2 * D:]), jnp.tile(be2, 8))
    ea2, ea2_p, partials = _sc_l2_call()(ps, pd, q_p, src, dst, zeros)
    x = _node_update(x, partials, Wn2[:D], Wn2[D:], bn2)

    cat = _edge_cat(ea0_p, ea1_p, ea2_p)
    return x, ea2, cat.reshape(E, 3 * DE)
